# Initial kernel scaffold; baseline (speedup 1.0000x reference)
import jax
import jax.numpy as jnp
from jax import lax
from jax.experimental import pallas as pl
from jax.experimental.pallas import tpu as pltpu
from jax.experimental.pallas import tpu_sc as plsc

VOCAB = 1000000
EMBED_DIM = 32
BATCH = 16384
HIST = 50

NC = 2
NS = 16
NW = NC * NS

TOTAL = BATCH * HIST
B_PER_W = TOTAL // NW
CHUNK = 1024
N_CHUNKS = B_PER_W // CHUNK


def _body(idx_hbm, table_hbm, out_hbm, idx_v, rows_v, sem):
  wid = lax.axis_index("s") * NC + lax.axis_index("c")
  base = wid * B_PER_W

  def step(g, carry):
    off = base + g * CHUNK
    pltpu.sync_copy(idx_hbm.at[pl.ds(off, CHUNK)], idx_v)
    pltpu.async_copy(table_hbm.at[idx_v], rows_v, sem).wait()
    pltpu.sync_copy(rows_v, out_hbm.at[pl.ds(off, CHUNK)])
    return carry

  lax.fori_loop(0, N_CHUNKS, step, 0)


@jax.jit
def _embed(x_flat, table):
  mesh = plsc.VectorSubcoreMesh(core_axis_name="c", subcore_axis_name="s")
  return pl.kernel(
      _body,
      out_type=jax.ShapeDtypeStruct((TOTAL, EMBED_DIM), jnp.float32),
      mesh=mesh,
      scratch_types=[
          pltpu.VMEM((CHUNK,), jnp.int32),
          pltpu.VMEM((CHUNK, EMBED_DIM), jnp.float32),
          pltpu.SemaphoreType.DMA,
      ],
  )(x_flat, table)


def kernel(x, table):
  x_flat = x.reshape(-1).astype(jnp.int32)
  out = _embed(x_flat, table)
  return out.reshape(BATCH, HIST, EMBED_DIM)


# speedup vs baseline: 1.0943x; 1.0943x over previous
import jax
import jax.numpy as jnp
from jax import lax
from jax.experimental import pallas as pl
from jax.experimental.pallas import tpu as pltpu
from jax.experimental.pallas import tpu_sc as plsc

VOCAB = 1000000
EMBED_DIM = 32
BATCH = 16384
HIST = 50

NC = 2
NS = 16
NW = NC * NS

TOTAL = BATCH * HIST
B_PER_W = TOTAL // NW
CHUNK = 1024
N_CHUNKS = B_PER_W // CHUNK


def _body(idx_hbm, table_hbm, out_hbm, idx_v, rows_v, sem):
  wid = lax.axis_index("s") * NC + lax.axis_index("c")
  base = wid * B_PER_W

  def step(g, carry):
    off = base + g * CHUNK
    pltpu.sync_copy(idx_hbm.at[pl.ds(off, CHUNK)], idx_v)
    pltpu.async_copy(table_hbm.at[idx_v], rows_v, sem).wait()
    pltpu.sync_copy(rows_v, out_hbm.at[pl.ds(off, CHUNK)])
    return carry

  lax.fori_loop(0, N_CHUNKS, step, 0)


@jax.jit
def _embed(x_flat, table):
  mesh = plsc.VectorSubcoreMesh(core_axis_name="c", subcore_axis_name="s")
  return pl.kernel(
      _body,
      out_type=jax.ShapeDtypeStruct((TOTAL, EMBED_DIM), jnp.float32),
      mesh=mesh,
      scratch_types=[
          pltpu.VMEM((CHUNK,), jnp.int32),
          pltpu.VMEM((CHUNK, EMBED_DIM), jnp.float32),
          pltpu.SemaphoreType.DMA,
      ],
      compiler_params=pltpu.CompilerParams(use_tc_tiling_on_sc=False),
  )(x_flat, table)


def kernel(x, table):
  x_flat = x.reshape(-1).astype(jnp.int32)
  out = _embed(x_flat, table)
  return out.reshape(BATCH, HIST, EMBED_DIM)


# speedup vs baseline: 1.1141x; 1.0180x over previous
import jax
import jax.numpy as jnp
from jax import lax
from jax.experimental import pallas as pl
from jax.experimental.pallas import tpu as pltpu
from jax.experimental.pallas import tpu_sc as plsc

VOCAB = 1000000
EMBED_DIM = 32
BATCH = 16384
HIST = 50

NC = 2
NS = 16
NW = NC * NS

TOTAL = BATCH * HIST
B_PER_W = TOTAL // NW
CHUNK = 640
N_CHUNKS = B_PER_W // CHUNK
NBUF = 4
LEAD = 2


def _body(idx_hbm, table_hbm, out_hbm, idx_v, rows_v, gsem, ssem):
  wid = lax.axis_index("s") * NC + lax.axis_index("c")
  base = wid * B_PER_W

  pltpu.sync_copy(idx_hbm.at[pl.ds(base, B_PER_W)], idx_v)

  def gather_desc(g, b):
    return pltpu.make_async_copy(
        table_hbm.at[idx_v.at[pl.ds(g * CHUNK, CHUNK)]], rows_v[b],
        gsem.at[b])

  def store_desc(g, b):
    return pltpu.make_async_copy(
        rows_v[b], out_hbm.at[pl.ds(base + g * CHUNK, CHUNK)], ssem.at[b])

  for b in range(LEAD):
    gather_desc(b, b).start()

  @pl.loop(0, N_CHUNKS, step=NBUF)
  def _ring(g0):
    for b in range(NBUF):
      g = g0 + b

      @pl.when(g < N_CHUNKS)
      def _():
        bn = (b + LEAD) % NBUF
        gn = g + LEAD

        @pl.when(gn < N_CHUNKS)
        def _():
          @pl.when(gn >= NBUF)
          def _():
            store_desc(gn - NBUF, bn).wait()

          gather_desc(gn, bn).start()

        gather_desc(g, b).wait()
        store_desc(g, b).start()

  for b in range(NBUF):
    g = N_CHUNKS - NBUF + b
    store_desc(g, b).wait()


@jax.jit
def _embed(x_flat, table):
  mesh = plsc.VectorSubcoreMesh(core_axis_name="c", subcore_axis_name="s")
  return pl.kernel(
      _body,
      out_type=jax.ShapeDtypeStruct((TOTAL, EMBED_DIM), jnp.float32),
      mesh=mesh,
      scratch_types=[
          pltpu.VMEM((B_PER_W,), jnp.int32),
          [pltpu.VMEM((CHUNK, EMBED_DIM), jnp.float32) for _ in range(NBUF)],
          pltpu.SemaphoreType.DMA((NBUF,)),
          pltpu.SemaphoreType.DMA((NBUF,)),
      ],
      compiler_params=pltpu.CompilerParams(use_tc_tiling_on_sc=False),
  )(x_flat, table)


def kernel(x, table):
  x_flat = x.reshape(-1).astype(jnp.int32)
  out = _embed(x_flat, table)
  return out.reshape(BATCH, HIST, EMBED_DIM)


# speedup vs baseline: 1.4516x; 1.3030x over previous
import jax
import jax.numpy as jnp
from jax import lax
from jax.experimental import pallas as pl
from jax.experimental.pallas import tpu as pltpu
from jax.experimental.pallas import tpu_sc as plsc

VOCAB = 1000000
EMBED_DIM = 32
BATCH = 16384
HIST = 50

NC = 2
NS = 16
NW = NC * NS

BB = BATCH // NW
L = 16


def _body(x_hbm, table_hbm, out_hbm, xblk, idxh, rows, cols,
          xsem, gsem, ssem):
  wid = lax.axis_index("s") * NC + lax.axis_index("c")
  b0 = wid * BB

  pltpu.make_async_copy(x_hbm.at[pl.ds(b0 * HIST, BB * HIST)], xblk,
                        xsem).start()

  lane = lax.iota(jnp.int32, L)
  lane_h = lane * HIST

  def build_idx(h, slot):
    def jstep(j, carry):
      v = plsc.load_gather(xblk, [lane_h + (j * (L * HIST) + h)])
      idxh[slot][pl.ds(j * L, L)] = v
      return carry
    lax.fori_loop(0, BB // L, jstep, 0, unroll=8)

  def gather_desc(slot):
    return pltpu.make_async_copy(
        table_hbm.at[idxh[slot]], rows[slot], gsem.at[slot])

  def store_desc(h, slot):
    return pltpu.make_async_copy(
        cols[slot], out_hbm.at[h, :, pl.ds(b0, BB)], ssem.at[slot])

  def transpose(slot):
    for e in range(EMBED_DIM):
      ev = jnp.full((L,), e, jnp.int32)

      def jstep(j, carry, ev=ev, slot=slot, e=e):
        v = plsc.load_gather(rows[slot], [lane + j * L, ev])
        cols[slot][e, pl.ds(j * L, L)] = v
        return carry
      lax.fori_loop(0, BB // L, jstep, 0, unroll=16)

  pltpu.make_async_copy(x_hbm.at[pl.ds(b0 * HIST, BB * HIST)], xblk,
                        xsem).wait()

  build_idx(0, 0)
  gather_desc(0).start()
  build_idx(1, 1)
  gather_desc(1).start()

  @pl.loop(0, HIST, step=2)
  def _hloop(h0):
    for s in range(2):
      h = h0 + s
      hn = h + 2
      gather_desc(s).wait()

      @pl.when(h >= 2)
      def _():
        store_desc(h - 2, s).wait()

      @pl.when(hn < HIST)
      def _():
        build_idx(hn, s)

      transpose(s)

      @pl.when(hn < HIST)
      def _():
        gather_desc(s).start()

      store_desc(h, s).start()

  store_desc(HIST - 2, 0).wait()
  store_desc(HIST - 1, 1).wait()


@jax.jit
def _embed(x_flat, table):
  mesh = plsc.VectorSubcoreMesh(core_axis_name="c", subcore_axis_name="s")
  return pl.kernel(
      _body,
      out_type=jax.ShapeDtypeStruct((HIST, EMBED_DIM, BATCH), jnp.float32),
      mesh=mesh,
      scratch_types=[
          pltpu.VMEM((BB * HIST,), jnp.int32),
          [pltpu.VMEM((BB,), jnp.int32) for _ in range(2)],
          [pltpu.VMEM((BB, EMBED_DIM), jnp.float32) for _ in range(2)],
          [pltpu.VMEM((EMBED_DIM, BB), jnp.float32) for _ in range(2)],
          pltpu.SemaphoreType.DMA,
          pltpu.SemaphoreType.DMA((2,)),
          pltpu.SemaphoreType.DMA((2,)),
      ],
      compiler_params=pltpu.CompilerParams(
          use_tc_tiling_on_sc=False, needs_layout_passes=False),
  )(x_flat, table)


def kernel(x, table):
  x_flat = x.reshape(-1).astype(jnp.int32)
  out = _embed(x_flat, table)
  return out.transpose(2, 0, 1)


# speedup vs baseline: 1.6624x; 1.1452x over previous
import jax
import jax.numpy as jnp
from jax import lax
from jax.experimental import pallas as pl
from jax.experimental.pallas import tpu as pltpu
from jax.experimental.pallas import tpu_sc as plsc

VOCAB = 1000000
EMBED_DIM = 32
BATCH = 16384
HIST = 50

NC = 2
NS = 16
NW = NC * NS

BB = BATCH // NW
L = 16


def _body(x_hbm, table_hbm, out_hbm, xblk, idxh, rows, cols,
          xsem, gsem, ssem):
  wid = lax.axis_index("s") * NC + lax.axis_index("c")
  b0 = wid * BB

  pltpu.make_async_copy(x_hbm.at[pl.ds(b0 * HIST, BB * HIST)], xblk,
                        xsem).start()

  lane = lax.iota(jnp.int32, L)
  lane_h = lane * HIST

  def build_idx(h, slot):
    @plsc.parallel_loop(0, BB // L, unroll=8)
    def jstep(j):
      v = plsc.load_gather(xblk, [lane_h + (j * (L * HIST) + h)])
      idxh[slot][pl.ds(j * L, L)] = v

  def gather_desc(slot):
    return pltpu.make_async_copy(
        table_hbm.at[idxh[slot]], rows[slot], gsem.at[slot])

  def store_desc(h, slot):
    return pltpu.make_async_copy(
        cols[slot], out_hbm.at[h, :, pl.ds(b0, BB)], ssem.at[slot])

  def transpose(slot):
    for e in range(EMBED_DIM):
      ev = jnp.full((L,), e, jnp.int32)

      @plsc.parallel_loop(0, BB // L, unroll=8)
      def jstep(j, ev=ev, slot=slot, e=e):
        v = plsc.load_gather(rows[slot], [lane + j * L, ev])
        cols[slot][e, pl.ds(j * L, L)] = v

  pltpu.make_async_copy(x_hbm.at[pl.ds(b0 * HIST, BB * HIST)], xblk,
                        xsem).wait()

  build_idx(0, 0)
  gather_desc(0).start()
  build_idx(1, 1)
  gather_desc(1).start()

  @pl.loop(0, HIST, step=2)
  def _hloop(h0):
    for s in range(2):
      h = h0 + s
      hn = h + 2
      gather_desc(s).wait()

      @pl.when(h >= 2)
      def _():
        store_desc(h - 2, s).wait()

      @pl.when(hn < HIST)
      def _():
        build_idx(hn, s)

      transpose(s)

      @pl.when(hn < HIST)
      def _():
        gather_desc(s).start()

      store_desc(h, s).start()

  store_desc(HIST - 2, 0).wait()
  store_desc(HIST - 1, 1).wait()


@jax.jit
def _embed(x_flat, table):
  mesh = plsc.VectorSubcoreMesh(core_axis_name="c", subcore_axis_name="s")
  return pl.kernel(
      _body,
      out_type=jax.ShapeDtypeStruct((HIST, EMBED_DIM, BATCH), jnp.float32),
      mesh=mesh,
      scratch_types=[
          pltpu.VMEM((BB * HIST,), jnp.int32),
          [pltpu.VMEM((BB,), jnp.int32) for _ in range(2)],
          [pltpu.VMEM((BB, EMBED_DIM), jnp.float32) for _ in range(2)],
          [pltpu.VMEM((EMBED_DIM, BB), jnp.float32) for _ in range(2)],
          pltpu.SemaphoreType.DMA,
          pltpu.SemaphoreType.DMA((2,)),
          pltpu.SemaphoreType.DMA((2,)),
      ],
      compiler_params=pltpu.CompilerParams(
          use_tc_tiling_on_sc=False, needs_layout_passes=False),
  )(x_flat, table)


def kernel(x, table):
  x_flat = x.reshape(-1).astype(jnp.int32)
  out = _embed(x_flat, table)
  return out.transpose(2, 0, 1)
